# baseline (speedup 1.0000x reference)
import jax
import jax.numpy as jnp
from jax import lax
from jax.experimental import pallas as pl
from jax.experimental.pallas import tpu as pltpu
from jax.experimental.pallas import tpu_sc as plsc

N = 10000
NP = 10240
E = 320000
D = 128
NEG_SLOPE = 0.2
NC = 2
NS = 16
NW = NC * NS
EPW = E // NW
CH = 80
NCHUNK = EPW // CH
BCH = 40
NCHB = EPW // BCH
GRP = EPW // 16
STRIPE = NP // NS
LPS = STRIPE // 16
RBLK = 512
KUNROLL = 128

_MESH = plsc.VectorSubcoreMesh(core_axis_name="c", subcore_axis_name="s")
_SC_PARAMS = pltpu.CompilerParams(needs_layout_passes=False)


def _mm_body(a_ref, b_ref, o_ref):
    o_ref[...] = jnp.dot(a_ref[...], b_ref[...],
                         preferred_element_type=jnp.float32)


def _matmul(feat_pad, wt):
    return pl.pallas_call(
        _mm_body,
        grid=(NP // RBLK,),
        in_specs=[pl.BlockSpec((RBLK, D), lambda i: (i, 0)),
                  pl.BlockSpec((D, D), lambda i: (0, 0))],
        out_specs=pl.BlockSpec((RBLK, D), lambda i: (i, 0)),
        out_shape=jax.ShapeDtypeStruct((NP, D), jnp.float32),
    )(feat_pad, wt)


def _comb_body(p0_ref, p1_ref, h_ref, t_ref):
    s = p0_ref[...] + p1_ref[...]
    h_ref[...] = s
    t_ref[...] = jnp.tanh(s)


def _combine(p0, p1):
    return pl.pallas_call(
        _comb_body,
        grid=(NP // RBLK,),
        in_specs=[pl.BlockSpec((RBLK, D), lambda i: (i, 0)),
                  pl.BlockSpec((RBLK, D), lambda i: (i, 0))],
        out_specs=[pl.BlockSpec((RBLK, D), lambda i: (i, 0)),
                   pl.BlockSpec((RBLK, D), lambda i: (i, 0))],
        out_shape=[jax.ShapeDtypeStruct((NP, D), jnp.float32),
                   jax.ShapeDtypeStruct((NP, D), jnp.float32)],
    )(p0, p1)


def _scatter_body(h0, src, dst_in, zeros, part,
                  srcb, d2, rows0, rows1, hacc, sem0, sem1):
    cid = lax.axis_index("c")
    sid = lax.axis_index("s")
    w = cid * NS + sid
    sl_stripe = pl.ds(sid * STRIPE, STRIPE)
    pltpu.sync_copy(src.at[pl.ds(w * EPW, EPW)], srcb)
    pltpu.sync_copy(dst_in.at[pl.ds(w * EPW, EPW)], d2)
    rows = (rows0, rows1)
    sems = (sem0, sem1)
    pltpu.async_copy(h0.at[srcb.at[pl.ds(0, CH)]], rows0, sem0)
    pltpu.async_copy(h0.at[srcb.at[pl.ds(CH, CH)]], rows1, sem1)
    pltpu.sync_copy(zeros.at[sl_stripe], hacc.at[sl_stripe])
    plsc.subcore_barrier()

    def pair(i, c):
        for b in range(2):
            g = 2 * i + b
            pltpu.make_async_copy(h0.at[pl.ds(0, CH)], rows[b], sems[b]).wait()
            pltpu.sync_copy(rows[b], hacc.at[d2.at[pl.ds(g * CH, CH)]],
                            add=True)
            gn = g + 2

            @pl.when(gn < NCHUNK)
            def _():
                pltpu.async_copy(h0.at[srcb.at[pl.ds(gn * CH, CH)]],
                                 rows[b], sems[b])
        return c

    lax.fori_loop(0, NCHUNK // 2, pair, 0)
    pltpu.make_async_copy(h0.at[pl.ds(0, CH)], rows0, sem0).wait()
    pltpu.sync_copy(rows0, hacc.at[d2.at[pl.ds((NCHUNK - 1) * CH, CH)]],
                    add=True)
    plsc.subcore_barrier()
    pltpu.sync_copy(hacc.at[sl_stripe], part.at[cid, sl_stripe])


_scatter_add = pl.kernel(
    _scatter_body,
    out_type=jax.ShapeDtypeStruct((NC, NP, D), jnp.float32),
    mesh=_MESH,
    compiler_params=_SC_PARAMS,
    scratch_types=[pltpu.VMEM((EPW,), jnp.int32),
                   pltpu.VMEM((EPW,), jnp.int32),
                   pltpu.VMEM((CH, D), jnp.float32),
                   pltpu.VMEM((CH, D), jnp.float32),
                   pltpu.VMEM_SHARED((NP, D), jnp.float32),
                   pltpu.SemaphoreType.DMA,
                   pltpu.SemaphoreType.DMA],
)


def _edge_body(h, t, src, dst, e_out, m_part,
               srcb, dstb, hrow0, hrow1, trow0, trow1,
               e_v, m_tile, macc, tmp, m_stage, hs0, hs1, ts0, ts1):
    cid = lax.axis_index("c")
    sid = lax.axis_index("s")
    w = cid * NS + sid
    pltpu.sync_copy(src.at[pl.ds(w * EPW, EPW)], srcb)
    pltpu.sync_copy(dst.at[pl.ds(w * EPW, EPW)], dstb)
    hrow = (hrow0, hrow1)
    trow = (trow0, trow1)
    hsem = (hs0, hs1)
    tsem = (ts0, ts1)

    def start(g, b):
        sl = pl.ds(g * CH, CH)
        pltpu.async_copy(h.at[srcb.at[sl]], hrow[b], hsem[b])
        pltpu.async_copy(t.at[dstb.at[sl]], trow[b], tsem[b])

    start(0, 0)
    start(1, 1)

    neginf = jnp.full((16,), -jnp.inf, jnp.float32)

    def init(i, c):
        m_tile[pl.ds(i * 16, 16)] = neginf
        return c

    lax.fori_loop(0, NP // 16, init, 0)

    rowiota = lax.iota(jnp.int32, 16)
    rot = [(rowiota + kk) & (KUNROLL - 1) for kk in range(KUNROLL)]
    rows_c = [rowiota + q * 16 for q in range(CH // 16)]

    def compute(g, b):
        for q in range(CH // 16):
            rowi = rows_c[q]

            def dotk(j, acc):
                kbase = j * KUNROLL
                for kk in range(KUNROLL):
                    kv = kbase + rot[kk]
                    a = plsc.load_gather(hrow[b], [rowi, kv])
                    bb = plsc.load_gather(trow[b], [rowi, kv])
                    acc = acc + a * bb
                return acc

            e16 = lax.fori_loop(0, D // KUNROLL, dotk,
                                jnp.zeros((16,), jnp.float32))
            e16 = jnp.where(e16 > 0, e16, NEG_SLOPE * e16)
            off = pl.ds(g * CH + q * 16, 16)
            e_v[off] = e16
            d16 = dstb[off]

            def mcond(p):
                return jnp.any(p)

            def mbody(p):
                cur = plsc.load_gather(m_tile, [d16])
                upd = jnp.logical_and(p, e16 > cur)
                plsc.store_scatter(m_tile, [d16], e16, mask=upd)
                cur2 = plsc.load_gather(m_tile, [d16])
                return e16 > cur2

            lax.while_loop(mcond, mbody, jnp.ones((16,), jnp.bool_))

    def pair(i, c):
        for b in range(2):
            g = 2 * i + b
            pltpu.make_async_copy(h.at[pl.ds(0, CH)], hrow[b], hsem[b]).wait()
            pltpu.make_async_copy(t.at[pl.ds(0, CH)], trow[b], tsem[b]).wait()
            compute(g, b)
            gn = g + 2

            @pl.when(gn < NCHUNK)
            def _():
                start(gn, b)
        return c

    lax.fori_loop(0, NCHUNK // 2, pair, 0)
    pltpu.make_async_copy(h.at[pl.ds(0, CH)], hrow0, hs0).wait()
    pltpu.make_async_copy(t.at[pl.ds(0, CH)], trow0, ts0).wait()
    compute(NCHUNK - 1, 0)
    pltpu.sync_copy(e_v, e_out.at[pl.ds(w * EPW, EPW)])

    pltpu.sync_copy(m_tile, m_stage.at[sid])
    plsc.subcore_barrier()
    sb = sid * STRIPE
    pltpu.sync_copy(m_stage.at[0, pl.ds(sb, STRIPE)], macc)

    def comb(ti, c):
        pltpu.sync_copy(m_stage.at[ti, pl.ds(sb, STRIPE)], tmp)

        def mx(j, c2):
            sl = pl.ds(j * 16, 16)
            macc[sl] = jnp.maximum(macc[sl], tmp[sl])
            return c2

        lax.fori_loop(0, LPS, mx, 0)
        return c

    lax.fori_loop(1, NS, comb, 0)
    pltpu.sync_copy(macc, m_part.at[cid, pl.ds(sb, STRIPE)])


_edge_dot = pl.kernel(
    _edge_body,
    out_type=(jax.ShapeDtypeStruct((E,), jnp.float32),
              jax.ShapeDtypeStruct((NC, NP), jnp.float32)),
    mesh=_MESH,
    compiler_params=_SC_PARAMS,
    scratch_types=[pltpu.VMEM((EPW,), jnp.int32),
                   pltpu.VMEM((EPW,), jnp.int32),
                   pltpu.VMEM((CH, D), jnp.float32),
                   pltpu.VMEM((CH, D), jnp.float32),
                   pltpu.VMEM((CH, D), jnp.float32),
                   pltpu.VMEM((CH, D), jnp.float32),
                   pltpu.VMEM((EPW,), jnp.float32),
                   pltpu.VMEM((NP,), jnp.float32),
                   pltpu.VMEM((STRIPE,), jnp.float32),
                   pltpu.VMEM((STRIPE,), jnp.float32),
                   pltpu.VMEM_SHARED((NS, NP), jnp.float32),
                   pltpu.SemaphoreType.DMA,
                   pltpu.SemaphoreType.DMA,
                   pltpu.SemaphoreType.DMA,
                   pltpu.SemaphoreType.DMA],
)


def _expsum_body(e_in, dst, m_part, eexp_out, s_part,
                 dstb, e_v, x_v, m_loc, s_tile, sacc, tmp, stage):
    cid = lax.axis_index("c")
    sid = lax.axis_index("s")
    w = cid * NS + sid
    pltpu.sync_copy(dst.at[pl.ds(w * EPW, EPW)], dstb)
    pltpu.sync_copy(e_in.at[pl.ds(w * EPW, EPW)], e_v)
    pltpu.sync_copy(m_part.at[0], m_loc)
    pltpu.sync_copy(m_part.at[1], s_tile)
    neginf = jnp.full((16,), -jnp.inf, jnp.float32)
    zeros16 = jnp.zeros((16,), jnp.float32)

    def initm(i, c):
        sl = pl.ds(i * 16, 16)
        v = jnp.maximum(m_loc[sl], s_tile[sl])
        v = jnp.where(v == neginf, zeros16, v)
        m_loc[sl] = v
        s_tile[sl] = zeros16
        return c

    lax.fori_loop(0, NP // 16, initm, 0)

    def grp(g, c):
        sl = pl.ds(g * 16, 16)
        d16 = dstb[sl]
        mm = plsc.load_gather(m_loc, [d16])
        x = jnp.exp(e_v[sl] - mm)
        x_v[sl] = x
        plsc.addupdate_scatter(s_tile, [d16], x)
        return c

    lax.fori_loop(0, GRP, grp, 0)
    pltpu.sync_copy(x_v, eexp_out.at[pl.ds(w * EPW, EPW)])

    pltpu.sync_copy(s_tile, stage.at[sid])
    plsc.subcore_barrier()
    sb = sid * STRIPE
    pltpu.sync_copy(stage.at[0, pl.ds(sb, STRIPE)], sacc)

    def comb(ti, c):
        pltpu.sync_copy(stage.at[ti, pl.ds(sb, STRIPE)], tmp)

        def ad(j, c2):
            sl = pl.ds(j * 16, 16)
            sacc[sl] = sacc[sl] + tmp[sl]
            return c2

        lax.fori_loop(0, LPS, ad, 0)
        return c

    lax.fori_loop(1, NS, comb, 0)
    pltpu.sync_copy(sacc, s_part.at[cid, pl.ds(sb, STRIPE)])


_expsum = pl.kernel(
    _expsum_body,
    out_type=(jax.ShapeDtypeStruct((E,), jnp.float32),
              jax.ShapeDtypeStruct((NC, NP), jnp.float32)),
    mesh=_MESH,
    compiler_params=_SC_PARAMS,
    scratch_types=[pltpu.VMEM((EPW,), jnp.int32),
                   pltpu.VMEM((EPW,), jnp.float32),
                   pltpu.VMEM((EPW,), jnp.float32),
                   pltpu.VMEM((NP,), jnp.float32),
                   pltpu.VMEM((NP,), jnp.float32),
                   pltpu.VMEM((STRIPE,), jnp.float32),
                   pltpu.VMEM((STRIPE,), jnp.float32),
                   pltpu.VMEM_SHARED((NS, NP), jnp.float32)],
)


def _norm_body(eexp_in, dst, s_part, out, dstb, x_v, o_v, s_loc, tmp_full):
    cid = lax.axis_index("c")
    sid = lax.axis_index("s")
    w = cid * NS + sid
    pltpu.sync_copy(dst.at[pl.ds(w * EPW, EPW)], dstb)
    pltpu.sync_copy(eexp_in.at[pl.ds(w * EPW, EPW)], x_v)
    pltpu.sync_copy(s_part.at[0], s_loc)
    pltpu.sync_copy(s_part.at[1], tmp_full)

    def inits(i, c):
        sl = pl.ds(i * 16, 16)
        s_loc[sl] = s_loc[sl] + tmp_full[sl]
        return c

    lax.fori_loop(0, NP // 16, inits, 0)

    def grp(g, c):
        sl = pl.ds(g * 16, 16)
        d16 = dstb[sl]
        ss = plsc.load_gather(s_loc, [d16])
        o_v[sl] = x_v[sl] / ss
        return c

    lax.fori_loop(0, GRP, grp, 0)
    pltpu.sync_copy(o_v, out.at[pl.ds(w * EPW, EPW)])


_normalize = pl.kernel(
    _norm_body,
    out_type=jax.ShapeDtypeStruct((E,), jnp.float32),
    mesh=_MESH,
    compiler_params=_SC_PARAMS,
    scratch_types=[pltpu.VMEM((EPW,), jnp.int32),
                   pltpu.VMEM((EPW,), jnp.float32),
                   pltpu.VMEM((EPW,), jnp.float32),
                   pltpu.VMEM((NP,), jnp.float32),
                   pltpu.VMEM((NP,), jnp.float32)],
)


def kernel(feat, edge_index, W):
    feat_pad = jnp.pad(feat, ((0, NP - N), (0, 0)))
    wt = W.T
    src = edge_index[0]
    dst = edge_index[1]
    zeros = jnp.zeros((NP, D), jnp.float32)
    h0 = _matmul(feat_pad, wt)
    part = _scatter_add(h0, src, dst, zeros)
    h_pad, t_pad = _combine(part[0], part[1])
    e, m_part = _edge_dot(h_pad, t_pad, src, dst)
    eexp, s_part = _expsum(e, dst, m_part)
    e_soft = _normalize(eexp, dst, s_part)
    return h_pad[:N], e_soft


# speedup vs baseline: 1.1939x; 1.1939x over previous
import jax
import jax.numpy as jnp
from jax import lax
from jax.experimental import pallas as pl
from jax.experimental.pallas import tpu as pltpu
from jax.experimental.pallas import tpu_sc as plsc

N = 10000
NP = 10240
E = 320000
D = 128
NEG_SLOPE = 0.2
NC = 2
NS = 16
NW = NC * NS
EPW = E // NW
CH = 80
NCHUNK = EPW // CH
BCH = 40
NCHB = EPW // BCH
GRP = EPW // 16
STRIPE = NP // NS
LPS = STRIPE // 16
RBLK = 512
KUNROLL = 32

_MESH = plsc.VectorSubcoreMesh(core_axis_name="c", subcore_axis_name="s")
_SC_PARAMS = pltpu.CompilerParams(needs_layout_passes=False)


def _mm_body(a_ref, b_ref, o_ref):
    o_ref[...] = jnp.dot(a_ref[...], b_ref[...],
                         preferred_element_type=jnp.float32)


def _matmul(feat_pad, wt):
    return pl.pallas_call(
        _mm_body,
        grid=(NP // RBLK,),
        in_specs=[pl.BlockSpec((RBLK, D), lambda i: (i, 0)),
                  pl.BlockSpec((D, D), lambda i: (0, 0))],
        out_specs=pl.BlockSpec((RBLK, D), lambda i: (i, 0)),
        out_shape=jax.ShapeDtypeStruct((NP, D), jnp.float32),
    )(feat_pad, wt)


def _comb_body(p0_ref, p1_ref, h_ref, t_ref):
    s = p0_ref[...] + p1_ref[...]
    h_ref[...] = s
    t_ref[...] = jnp.tanh(s)


def _combine(p0, p1):
    return pl.pallas_call(
        _comb_body,
        grid=(NP // RBLK,),
        in_specs=[pl.BlockSpec((RBLK, D), lambda i: (i, 0)),
                  pl.BlockSpec((RBLK, D), lambda i: (i, 0))],
        out_specs=[pl.BlockSpec((RBLK, D), lambda i: (i, 0)),
                   pl.BlockSpec((RBLK, D), lambda i: (i, 0))],
        out_shape=[jax.ShapeDtypeStruct((NP, D), jnp.float32),
                   jax.ShapeDtypeStruct((NP, D), jnp.float32)],
    )(p0, p1)


def _scatter_body(h0, src, dst_in, zeros, part,
                  srcb, d2, rows0, rows1, hacc, sem0, sem1):
    cid = lax.axis_index("c")
    sid = lax.axis_index("s")
    w = cid * NS + sid
    sl_stripe = pl.ds(sid * STRIPE, STRIPE)
    pltpu.sync_copy(src.at[pl.ds(w * EPW, EPW)], srcb)
    pltpu.sync_copy(dst_in.at[pl.ds(w * EPW, EPW)], d2)
    rows = (rows0, rows1)
    sems = (sem0, sem1)
    pltpu.async_copy(h0.at[srcb.at[pl.ds(0, CH)]], rows0, sem0)
    pltpu.async_copy(h0.at[srcb.at[pl.ds(CH, CH)]], rows1, sem1)
    pltpu.sync_copy(zeros.at[sl_stripe], hacc.at[sl_stripe])
    plsc.subcore_barrier()

    def pair(i, c):
        for b in range(2):
            g = 2 * i + b
            pltpu.make_async_copy(h0.at[pl.ds(0, CH)], rows[b], sems[b]).wait()
            pltpu.sync_copy(rows[b], hacc.at[d2.at[pl.ds(g * CH, CH)]],
                            add=True)
            gn = g + 2

            @pl.when(gn < NCHUNK)
            def _():
                pltpu.async_copy(h0.at[srcb.at[pl.ds(gn * CH, CH)]],
                                 rows[b], sems[b])
        return c

    lax.fori_loop(0, NCHUNK // 2, pair, 0)
    pltpu.make_async_copy(h0.at[pl.ds(0, CH)], rows0, sem0).wait()
    pltpu.sync_copy(rows0, hacc.at[d2.at[pl.ds((NCHUNK - 1) * CH, CH)]],
                    add=True)
    plsc.subcore_barrier()
    pltpu.sync_copy(hacc.at[sl_stripe], part.at[cid, sl_stripe])


_scatter_add = pl.kernel(
    _scatter_body,
    out_type=jax.ShapeDtypeStruct((NC, NP, D), jnp.float32),
    mesh=_MESH,
    compiler_params=_SC_PARAMS,
    scratch_types=[pltpu.VMEM((EPW,), jnp.int32),
                   pltpu.VMEM((EPW,), jnp.int32),
                   pltpu.VMEM((CH, D), jnp.float32),
                   pltpu.VMEM((CH, D), jnp.float32),
                   pltpu.VMEM_SHARED((NP, D), jnp.float32),
                   pltpu.SemaphoreType.DMA,
                   pltpu.SemaphoreType.DMA],
)


def _edge_body(h, t, src, dst, e_out, m_part,
               srcb, dstb, hrow0, hrow1, hrow2, trow0, trow1, trow2,
               e_v, m_tile, macc, tmp, m_stage, hs0, hs1, hs2, ts0, ts1, ts2):
    cid = lax.axis_index("c")
    sid = lax.axis_index("s")
    w = cid * NS + sid
    pltpu.sync_copy(src.at[pl.ds(w * EPW, EPW)], srcb)
    pltpu.sync_copy(dst.at[pl.ds(w * EPW, EPW)], dstb)
    hrow = (hrow0, hrow1, hrow2)
    trow = (trow0, trow1, trow2)
    hsem = (hs0, hs1, hs2)
    tsem = (ts0, ts1, ts2)

    def start(g, b):
        sl = pl.ds(g * CH, CH)
        pltpu.async_copy(h.at[srcb.at[sl]], hrow[b], hsem[b])
        pltpu.async_copy(t.at[dstb.at[sl]], trow[b], tsem[b])

    start(0, 0)
    start(1, 1)

    neginf = jnp.full((16,), -jnp.inf, jnp.float32)

    def init(i, c):
        m_tile[pl.ds(i * 16, 16)] = neginf
        return c

    lax.fori_loop(0, NP // 16, init, 0)

    rowiota = lax.iota(jnp.int32, 16)
    rot = [(rowiota + kk) & (KUNROLL - 1) for kk in range(KUNROLL)]
    rows_c = [rowiota + q * 16 for q in range(CH // 16)]

    def compute(g, b):
        for q in range(CH // 16):
            rowi = rows_c[q]

            def dotk(j, acc):
                kbase = j * KUNROLL
                for kk in range(KUNROLL):
                    kv = kbase + rot[kk]
                    a = plsc.load_gather(hrow[b], [rowi, kv])
                    bb = plsc.load_gather(trow[b], [rowi, kv])
                    acc = acc + a * bb
                return acc

            e16 = lax.fori_loop(0, D // KUNROLL, dotk,
                                jnp.zeros((16,), jnp.float32))
            e16 = jnp.where(e16 > 0, e16, NEG_SLOPE * e16)
            off = pl.ds(g * CH + q * 16, 16)
            e_v[off] = e16
            d16 = dstb[off]

            def mcond(p):
                return jnp.any(p)

            def mbody(p):
                cur = plsc.load_gather(m_tile, [d16])
                upd = jnp.logical_and(p, e16 > cur)
                plsc.store_scatter(m_tile, [d16], e16, mask=upd)
                cur2 = plsc.load_gather(m_tile, [d16])
                return e16 > cur2

            lax.while_loop(mcond, mbody, jnp.ones((16,), jnp.bool_))

    def triple(i, c):
        for b in range(3):
            g = 3 * i + b
            b2 = (b + 2) % 3
            pltpu.make_async_copy(h.at[pl.ds(0, CH)], hrow[b], hsem[b]).wait()
            pltpu.make_async_copy(t.at[pl.ds(0, CH)], trow[b], tsem[b]).wait()
            gn = g + 2

            @pl.when(gn < NCHUNK)
            def _():
                start(gn, b2)

            compute(g, b)
        return c

    lax.fori_loop(0, NCHUNK // 3, triple, 0)
    pltpu.make_async_copy(h.at[pl.ds(0, CH)], hrow0, hs0).wait()
    pltpu.make_async_copy(t.at[pl.ds(0, CH)], trow0, ts0).wait()
    compute(NCHUNK - 2, 0)
    pltpu.make_async_copy(h.at[pl.ds(0, CH)], hrow1, hs1).wait()
    pltpu.make_async_copy(t.at[pl.ds(0, CH)], trow1, ts1).wait()
    compute(NCHUNK - 1, 1)
    pltpu.sync_copy(e_v, e_out.at[pl.ds(w * EPW, EPW)])

    pltpu.sync_copy(m_tile, m_stage.at[sid])
    plsc.subcore_barrier()
    sb = sid * STRIPE
    pltpu.sync_copy(m_stage.at[0, pl.ds(sb, STRIPE)], macc)

    def comb(ti, c):
        pltpu.sync_copy(m_stage.at[ti, pl.ds(sb, STRIPE)], tmp)

        def mx(j, c2):
            sl = pl.ds(j * 16, 16)
            macc[sl] = jnp.maximum(macc[sl], tmp[sl])
            return c2

        lax.fori_loop(0, LPS, mx, 0)
        return c

    lax.fori_loop(1, NS, comb, 0)
    pltpu.sync_copy(macc, m_part.at[cid, pl.ds(sb, STRIPE)])


_edge_dot = pl.kernel(
    _edge_body,
    out_type=(jax.ShapeDtypeStruct((E,), jnp.float32),
              jax.ShapeDtypeStruct((NC, NP), jnp.float32)),
    mesh=_MESH,
    compiler_params=_SC_PARAMS,
    scratch_types=[pltpu.VMEM((EPW,), jnp.int32),
                   pltpu.VMEM((EPW,), jnp.int32),
                   pltpu.VMEM((CH, D), jnp.float32),
                   pltpu.VMEM((CH, D), jnp.float32),
                   pltpu.VMEM((CH, D), jnp.float32),
                   pltpu.VMEM((CH, D), jnp.float32),
                   pltpu.VMEM((CH, D), jnp.float32),
                   pltpu.VMEM((CH, D), jnp.float32),
                   pltpu.VMEM((EPW,), jnp.float32),
                   pltpu.VMEM((NP,), jnp.float32),
                   pltpu.VMEM((STRIPE,), jnp.float32),
                   pltpu.VMEM((STRIPE,), jnp.float32),
                   pltpu.VMEM_SHARED((NS, NP), jnp.float32),
                   pltpu.SemaphoreType.DMA,
                   pltpu.SemaphoreType.DMA,
                   pltpu.SemaphoreType.DMA,
                   pltpu.SemaphoreType.DMA,
                   pltpu.SemaphoreType.DMA,
                   pltpu.SemaphoreType.DMA],
)


def _expsum_body(e_in, dst, m_part, eexp_out, s_part,
                 dstb, e_v, x_v, m_loc, s_tile, sacc, tmp, stage):
    cid = lax.axis_index("c")
    sid = lax.axis_index("s")
    w = cid * NS + sid
    pltpu.sync_copy(dst.at[pl.ds(w * EPW, EPW)], dstb)
    pltpu.sync_copy(e_in.at[pl.ds(w * EPW, EPW)], e_v)
    pltpu.sync_copy(m_part.at[0], m_loc)
    pltpu.sync_copy(m_part.at[1], s_tile)
    neginf = jnp.full((16,), -jnp.inf, jnp.float32)
    zeros16 = jnp.zeros((16,), jnp.float32)

    def initm(i, c):
        sl = pl.ds(i * 16, 16)
        v = jnp.maximum(m_loc[sl], s_tile[sl])
        v = jnp.where(v == neginf, zeros16, v)
        m_loc[sl] = v
        s_tile[sl] = zeros16
        return c

    lax.fori_loop(0, NP // 16, initm, 0)

    def grp(g, c):
        sl = pl.ds(g * 16, 16)
        d16 = dstb[sl]
        mm = plsc.load_gather(m_loc, [d16])
        x = jnp.exp(e_v[sl] - mm)
        x_v[sl] = x
        plsc.addupdate_scatter(s_tile, [d16], x)
        return c

    lax.fori_loop(0, GRP, grp, 0)
    pltpu.sync_copy(x_v, eexp_out.at[pl.ds(w * EPW, EPW)])

    pltpu.sync_copy(s_tile, stage.at[sid])
    plsc.subcore_barrier()
    sb = sid * STRIPE
    pltpu.sync_copy(stage.at[0, pl.ds(sb, STRIPE)], sacc)

    def comb(ti, c):
        pltpu.sync_copy(stage.at[ti, pl.ds(sb, STRIPE)], tmp)

        def ad(j, c2):
            sl = pl.ds(j * 16, 16)
            sacc[sl] = sacc[sl] + tmp[sl]
            return c2

        lax.fori_loop(0, LPS, ad, 0)
        return c

    lax.fori_loop(1, NS, comb, 0)
    pltpu.sync_copy(sacc, s_part.at[cid, pl.ds(sb, STRIPE)])


_expsum = pl.kernel(
    _expsum_body,
    out_type=(jax.ShapeDtypeStruct((E,), jnp.float32),
              jax.ShapeDtypeStruct((NC, NP), jnp.float32)),
    mesh=_MESH,
    compiler_params=_SC_PARAMS,
    scratch_types=[pltpu.VMEM((EPW,), jnp.int32),
                   pltpu.VMEM((EPW,), jnp.float32),
                   pltpu.VMEM((EPW,), jnp.float32),
                   pltpu.VMEM((NP,), jnp.float32),
                   pltpu.VMEM((NP,), jnp.float32),
                   pltpu.VMEM((STRIPE,), jnp.float32),
                   pltpu.VMEM((STRIPE,), jnp.float32),
                   pltpu.VMEM_SHARED((NS, NP), jnp.float32)],
)


def _norm_body(eexp_in, dst, s_part, out, dstb, x_v, o_v, s_loc, tmp_full):
    cid = lax.axis_index("c")
    sid = lax.axis_index("s")
    w = cid * NS + sid
    pltpu.sync_copy(dst.at[pl.ds(w * EPW, EPW)], dstb)
    pltpu.sync_copy(eexp_in.at[pl.ds(w * EPW, EPW)], x_v)
    pltpu.sync_copy(s_part.at[0], s_loc)
    pltpu.sync_copy(s_part.at[1], tmp_full)

    def inits(i, c):
        sl = pl.ds(i * 16, 16)
        s_loc[sl] = s_loc[sl] + tmp_full[sl]
        return c

    lax.fori_loop(0, NP // 16, inits, 0)

    def grp(g, c):
        sl = pl.ds(g * 16, 16)
        d16 = dstb[sl]
        ss = plsc.load_gather(s_loc, [d16])
        o_v[sl] = x_v[sl] / ss
        return c

    lax.fori_loop(0, GRP, grp, 0)
    pltpu.sync_copy(o_v, out.at[pl.ds(w * EPW, EPW)])


_normalize = pl.kernel(
    _norm_body,
    out_type=jax.ShapeDtypeStruct((E,), jnp.float32),
    mesh=_MESH,
    compiler_params=_SC_PARAMS,
    scratch_types=[pltpu.VMEM((EPW,), jnp.int32),
                   pltpu.VMEM((EPW,), jnp.float32),
                   pltpu.VMEM((EPW,), jnp.float32),
                   pltpu.VMEM((NP,), jnp.float32),
                   pltpu.VMEM((NP,), jnp.float32)],
)


def kernel(feat, edge_index, W):
    feat_pad = jnp.pad(feat, ((0, NP - N), (0, 0)))
    wt = W.T
    src = edge_index[0]
    dst = edge_index[1]
    zeros = jnp.zeros((NP, D), jnp.float32)
    h0 = _matmul(feat_pad, wt)
    part = _scatter_add(h0, src, dst, zeros)
    h_pad, t_pad = _combine(part[0], part[1])
    e, m_part = _edge_dot(h_pad, t_pad, src, dst)
    eexp, s_part = _expsum(e, dst, m_part)
    e_soft = _normalize(eexp, dst, s_part)
    return h_pad[:N], e_soft


# speedup vs baseline: 1.2290x; 1.0293x over previous
import jax
import jax.numpy as jnp
from jax import lax
from jax.experimental import pallas as pl
from jax.experimental.pallas import tpu as pltpu
from jax.experimental.pallas import tpu_sc as plsc

N = 10000
NP = 10240
E = 320000
D = 128
NEG_SLOPE = 0.2
NC = 2
NS = 16
NW = NC * NS
EPW = E // NW
CH = 80
NCHUNK = EPW // CH
BCH = 40
NCHB = EPW // BCH
GRP = EPW // 16
STRIPE = NP // NS
LPS = STRIPE // 16
RBLK = 512
KUNROLL = 32

_MESH = plsc.VectorSubcoreMesh(core_axis_name="c", subcore_axis_name="s")
_SC_PARAMS = pltpu.CompilerParams(needs_layout_passes=False)


def _mm_body(a_ref, b_ref, o_ref):
    o_ref[...] = jnp.dot(a_ref[...], b_ref[...],
                         preferred_element_type=jnp.float32)


def _matmul(feat_pad, wt):
    return pl.pallas_call(
        _mm_body,
        grid=(NP // RBLK,),
        in_specs=[pl.BlockSpec((RBLK, D), lambda i: (i, 0)),
                  pl.BlockSpec((D, D), lambda i: (0, 0))],
        out_specs=pl.BlockSpec((RBLK, D), lambda i: (i, 0)),
        out_shape=jax.ShapeDtypeStruct((NP, D), jnp.float32),
    )(feat_pad, wt)


def _comb_body(p0_ref, p1_ref, h_ref, t_ref):
    s = p0_ref[...] + p1_ref[...]
    h_ref[...] = s
    t_ref[...] = jnp.tanh(s)


def _combine(p0, p1):
    return pl.pallas_call(
        _comb_body,
        grid=(NP // RBLK,),
        in_specs=[pl.BlockSpec((RBLK, D), lambda i: (i, 0)),
                  pl.BlockSpec((RBLK, D), lambda i: (i, 0))],
        out_specs=[pl.BlockSpec((RBLK, D), lambda i: (i, 0)),
                   pl.BlockSpec((RBLK, D), lambda i: (i, 0))],
        out_shape=[jax.ShapeDtypeStruct((NP, D), jnp.float32),
                   jax.ShapeDtypeStruct((NP, D), jnp.float32)],
    )(p0, p1)


def _scatter_body(h0, src, dst_in, zeros, part,
                  srcb, d2, rows0, rows1, hacc, sem0, sem1):
    cid = lax.axis_index("c")
    sid = lax.axis_index("s")
    w = cid * NS + sid
    sl_stripe = pl.ds(sid * STRIPE, STRIPE)
    pltpu.sync_copy(src.at[pl.ds(w * EPW, EPW)], srcb)
    pltpu.sync_copy(dst_in.at[pl.ds(w * EPW, EPW)], d2)
    rows = (rows0, rows1)
    sems = (sem0, sem1)
    pltpu.async_copy(h0.at[srcb.at[pl.ds(0, CH)]], rows0, sem0)
    pltpu.async_copy(h0.at[srcb.at[pl.ds(CH, CH)]], rows1, sem1)
    pltpu.sync_copy(zeros.at[sl_stripe], hacc.at[sl_stripe])
    plsc.subcore_barrier()

    def pair(i, c):
        for b in range(2):
            g = 2 * i + b
            pltpu.make_async_copy(h0.at[pl.ds(0, CH)], rows[b], sems[b]).wait()
            pltpu.sync_copy(rows[b], hacc.at[d2.at[pl.ds(g * CH, CH)]],
                            add=True)
            gn = g + 2

            @pl.when(gn < NCHUNK)
            def _():
                pltpu.async_copy(h0.at[srcb.at[pl.ds(gn * CH, CH)]],
                                 rows[b], sems[b])
        return c

    lax.fori_loop(0, NCHUNK // 2, pair, 0)
    pltpu.make_async_copy(h0.at[pl.ds(0, CH)], rows0, sem0).wait()
    pltpu.sync_copy(rows0, hacc.at[d2.at[pl.ds((NCHUNK - 1) * CH, CH)]],
                    add=True)
    plsc.subcore_barrier()
    pltpu.sync_copy(hacc.at[sl_stripe], part.at[cid, sl_stripe])


_scatter_add = pl.kernel(
    _scatter_body,
    out_type=jax.ShapeDtypeStruct((NC, NP, D), jnp.float32),
    mesh=_MESH,
    compiler_params=_SC_PARAMS,
    scratch_types=[pltpu.VMEM((EPW,), jnp.int32),
                   pltpu.VMEM((EPW,), jnp.int32),
                   pltpu.VMEM((CH, D), jnp.float32),
                   pltpu.VMEM((CH, D), jnp.float32),
                   pltpu.VMEM_SHARED((NP, D), jnp.float32),
                   pltpu.SemaphoreType.DMA,
                   pltpu.SemaphoreType.DMA],
)


def _edge_body(h, t, src, dst, e_out, m_part,
               srcb, dstb, hrow0, hrow1, trow0, trow1,
               e_v, m_tile, macc, tmp, m_stage, hs0, hs1, ts0, ts1):
    cid = lax.axis_index("c")
    sid = lax.axis_index("s")
    w = cid * NS + sid
    pltpu.sync_copy(src.at[pl.ds(w * EPW, EPW)], srcb)
    pltpu.sync_copy(dst.at[pl.ds(w * EPW, EPW)], dstb)
    hrow = (hrow0, hrow1)
    trow = (trow0, trow1)
    hsem = (hs0, hs1)
    tsem = (ts0, ts1)

    def start(g, b):
        sl = pl.ds(g * CH, CH)
        pltpu.async_copy(h.at[srcb.at[sl]], hrow[b], hsem[b])
        pltpu.async_copy(t.at[dstb.at[sl]], trow[b], tsem[b])

    start(0, 0)
    start(1, 1)

    neginf = jnp.full((16,), -jnp.inf, jnp.float32)

    def init(i, c):
        m_tile[pl.ds(i * 16, 16)] = neginf
        return c

    lax.fori_loop(0, NP // 16, init, 0)

    rowiota = lax.iota(jnp.int32, 16)
    rot = [(rowiota + kk) & (KUNROLL - 1) for kk in range(KUNROLL)]
    rows_c = [rowiota + q * 16 for q in range(CH // 16)]

    def compute(g, b):
        for q in range(CH // 16):
            rowi = rows_c[q]

            def dotk(j, acc):
                kbase = j * KUNROLL
                for kk in range(KUNROLL):
                    kv = kbase + rot[kk]
                    a = plsc.load_gather(hrow[b], [rowi, kv])
                    bb = plsc.load_gather(trow[b], [rowi, kv])
                    acc = acc + a * bb
                return acc

            e16 = lax.fori_loop(0, D // KUNROLL, dotk,
                                jnp.zeros((16,), jnp.float32))
            e16 = jnp.where(e16 > 0, e16, NEG_SLOPE * e16)
            off = pl.ds(g * CH + q * 16, 16)
            e_v[off] = e16
            d16 = dstb[off]

            def mcond(p):
                return jnp.any(p)

            def mbody(p):
                cur = plsc.load_gather(m_tile, [d16])
                upd = jnp.logical_and(p, e16 > cur)
                plsc.store_scatter(m_tile, [d16], e16, mask=upd)
                cur2 = plsc.load_gather(m_tile, [d16])
                return e16 > cur2

            lax.while_loop(mcond, mbody, jnp.ones((16,), jnp.bool_))

    def pair(i, c):
        for b in range(2):
            g = 2 * i + b
            pltpu.make_async_copy(h.at[pl.ds(0, CH)], hrow[b], hsem[b]).wait()
            pltpu.make_async_copy(t.at[pl.ds(0, CH)], trow[b], tsem[b]).wait()
            compute(g, b)
            gn = g + 2

            @pl.when(gn < NCHUNK)
            def _():
                start(gn, b)
        return c

    lax.fori_loop(0, NCHUNK // 2, pair, 0)
    pltpu.make_async_copy(h.at[pl.ds(0, CH)], hrow0, hs0).wait()
    pltpu.make_async_copy(t.at[pl.ds(0, CH)], trow0, ts0).wait()
    compute(NCHUNK - 1, 0)
    pltpu.sync_copy(e_v, e_out.at[pl.ds(w * EPW, EPW)])

    pltpu.sync_copy(m_tile, m_stage.at[sid])
    plsc.subcore_barrier()
    sb = sid * STRIPE
    pltpu.sync_copy(m_stage.at[0, pl.ds(sb, STRIPE)], macc)

    def comb(ti, c):
        pltpu.sync_copy(m_stage.at[ti, pl.ds(sb, STRIPE)], tmp)

        def mx(j, c2):
            sl = pl.ds(j * 16, 16)
            macc[sl] = jnp.maximum(macc[sl], tmp[sl])
            return c2

        lax.fori_loop(0, LPS, mx, 0)
        return c

    lax.fori_loop(1, NS, comb, 0)
    pltpu.sync_copy(macc, m_part.at[cid, pl.ds(sb, STRIPE)])


_edge_dot = pl.kernel(
    _edge_body,
    out_type=(jax.ShapeDtypeStruct((E,), jnp.float32),
              jax.ShapeDtypeStruct((NC, NP), jnp.float32)),
    mesh=_MESH,
    compiler_params=_SC_PARAMS,
    scratch_types=[pltpu.VMEM((EPW,), jnp.int32),
                   pltpu.VMEM((EPW,), jnp.int32),
                   pltpu.VMEM((CH, D), jnp.float32),
                   pltpu.VMEM((CH, D), jnp.float32),
                   pltpu.VMEM((CH, D), jnp.float32),
                   pltpu.VMEM((CH, D), jnp.float32),
                   pltpu.VMEM((EPW,), jnp.float32),
                   pltpu.VMEM((NP,), jnp.float32),
                   pltpu.VMEM((STRIPE,), jnp.float32),
                   pltpu.VMEM((STRIPE,), jnp.float32),
                   pltpu.VMEM_SHARED((NS, NP), jnp.float32),
                   pltpu.SemaphoreType.DMA,
                   pltpu.SemaphoreType.DMA,
                   pltpu.SemaphoreType.DMA,
                   pltpu.SemaphoreType.DMA],
)


def _expsum_body(e_in, dst, m_part, eexp_out, s_part,
                 dstb, e_v, x_v, m_loc, s_tile, sacc, tmp, stage):
    cid = lax.axis_index("c")
    sid = lax.axis_index("s")
    w = cid * NS + sid
    pltpu.sync_copy(dst.at[pl.ds(w * EPW, EPW)], dstb)
    pltpu.sync_copy(e_in.at[pl.ds(w * EPW, EPW)], e_v)
    pltpu.sync_copy(m_part.at[0], m_loc)
    pltpu.sync_copy(m_part.at[1], s_tile)
    neginf = jnp.full((16,), -jnp.inf, jnp.float32)
    zeros16 = jnp.zeros((16,), jnp.float32)

    def initm(i, c):
        sl = pl.ds(i * 16, 16)
        v = jnp.maximum(m_loc[sl], s_tile[sl])
        v = jnp.where(v == neginf, zeros16, v)
        m_loc[sl] = v
        s_tile[sl] = zeros16
        return c

    lax.fori_loop(0, NP // 16, initm, 0)

    def grp(g, c):
        sl = pl.ds(g * 16, 16)
        d16 = dstb[sl]
        mm = plsc.load_gather(m_loc, [d16])
        x = jnp.exp(e_v[sl] - mm)
        x_v[sl] = x
        plsc.addupdate_scatter(s_tile, [d16], x)
        return c

    lax.fori_loop(0, GRP, grp, 0)
    pltpu.sync_copy(x_v, eexp_out.at[pl.ds(w * EPW, EPW)])

    pltpu.sync_copy(s_tile, stage.at[sid])
    plsc.subcore_barrier()
    sb = sid * STRIPE
    pltpu.sync_copy(stage.at[0, pl.ds(sb, STRIPE)], sacc)

    def comb(ti, c):
        pltpu.sync_copy(stage.at[ti, pl.ds(sb, STRIPE)], tmp)

        def ad(j, c2):
            sl = pl.ds(j * 16, 16)
            sacc[sl] = sacc[sl] + tmp[sl]
            return c2

        lax.fori_loop(0, LPS, ad, 0)
        return c

    lax.fori_loop(1, NS, comb, 0)
    pltpu.sync_copy(sacc, s_part.at[cid, pl.ds(sb, STRIPE)])


_expsum = pl.kernel(
    _expsum_body,
    out_type=(jax.ShapeDtypeStruct((E,), jnp.float32),
              jax.ShapeDtypeStruct((NC, NP), jnp.float32)),
    mesh=_MESH,
    compiler_params=_SC_PARAMS,
    scratch_types=[pltpu.VMEM((EPW,), jnp.int32),
                   pltpu.VMEM((EPW,), jnp.float32),
                   pltpu.VMEM((EPW,), jnp.float32),
                   pltpu.VMEM((NP,), jnp.float32),
                   pltpu.VMEM((NP,), jnp.float32),
                   pltpu.VMEM((STRIPE,), jnp.float32),
                   pltpu.VMEM((STRIPE,), jnp.float32),
                   pltpu.VMEM_SHARED((NS, NP), jnp.float32)],
)


def _norm_body(eexp_in, dst, s_part, out, dstb, x_v, o_v, s_loc, tmp_full):
    cid = lax.axis_index("c")
    sid = lax.axis_index("s")
    w = cid * NS + sid
    pltpu.sync_copy(dst.at[pl.ds(w * EPW, EPW)], dstb)
    pltpu.sync_copy(eexp_in.at[pl.ds(w * EPW, EPW)], x_v)
    pltpu.sync_copy(s_part.at[0], s_loc)
    pltpu.sync_copy(s_part.at[1], tmp_full)

    def inits(i, c):
        sl = pl.ds(i * 16, 16)
        s_loc[sl] = s_loc[sl] + tmp_full[sl]
        return c

    lax.fori_loop(0, NP // 16, inits, 0)

    def grp(g, c):
        sl = pl.ds(g * 16, 16)
        d16 = dstb[sl]
        ss = plsc.load_gather(s_loc, [d16])
        o_v[sl] = x_v[sl] / ss
        return c

    lax.fori_loop(0, GRP, grp, 0)
    pltpu.sync_copy(o_v, out.at[pl.ds(w * EPW, EPW)])


_normalize = pl.kernel(
    _norm_body,
    out_type=jax.ShapeDtypeStruct((E,), jnp.float32),
    mesh=_MESH,
    compiler_params=_SC_PARAMS,
    scratch_types=[pltpu.VMEM((EPW,), jnp.int32),
                   pltpu.VMEM((EPW,), jnp.float32),
                   pltpu.VMEM((EPW,), jnp.float32),
                   pltpu.VMEM((NP,), jnp.float32),
                   pltpu.VMEM((NP,), jnp.float32)],
)


def kernel(feat, edge_index, W):
    feat_pad = jnp.pad(feat, ((0, NP - N), (0, 0)))
    wt = W.T
    src = edge_index[0]
    dst = edge_index[1]
    zeros = jnp.zeros((NP, D), jnp.float32)
    h0 = _matmul(feat_pad, wt)
    part = _scatter_add(h0, src, dst, zeros)
    h_pad, t_pad = _combine(part[0], part[1])
    e, m_part = _edge_dot(h_pad, t_pad, src, dst)
    eexp, s_part = _expsum(e, dst, m_part)
    e_soft = _normalize(eexp, dst, s_part)
    return h_pad[:N], e_soft
